# baseline (speedup 1.0000x reference)
import jax
import jax.numpy as jnp
from jax.experimental import pallas as pl
from jax.experimental.pallas import tpu as pltpu

_N_E = 1024
_E_DIM = 256
_P = 1024
_S = 4
_CS = _P // _S


def _compute(zbuf, obuf, buf, cb):
    z_b = zbuf[buf]
    enorm = jnp.sum(cb * cb, axis=1, keepdims=True)
    znorm = jnp.sum(z_b * z_b, axis=0, keepdims=True)
    mm2 = jax.lax.dot_general(
        cb * -2.0, z_b, (((1,), (0,)), ((), ())),
        preferred_element_type=jnp.float32)
    d = (znorm + enorm) + mm2
    iota_k = jax.lax.broadcasted_iota(jnp.int32, (_N_E, _P), 0)
    dmin = jnp.min(d, axis=0, keepdims=True)
    ikey = jnp.where(d == dmin, iota_k, _N_E)
    idx = jnp.min(ikey, axis=0, keepdims=True)
    onehot = (ikey == idx).astype(jnp.float32)
    obuf[buf] = jax.lax.dot_general(
        cb, onehot, (((0,), (0,)), ((), ())),
        preferred_element_type=jnp.float32)


def _vq_body(z_hbm, cb_ref, out_hbm, zbuf, obuf, in_sems, out_sems):
    t = pl.program_id(0)
    nb = pl.num_programs(0) - 1

    def in_copy(b, s):
        return pltpu.make_async_copy(
            z_hbm.at[b, :, pl.ds(s * _CS, _CS)],
            zbuf.at[jax.lax.rem(b, 2), :, pl.ds(s * _CS, _CS)],
            in_sems.at[jax.lax.rem(b, 2), s])

    def out_copy(b, s):
        return pltpu.make_async_copy(
            obuf.at[jax.lax.rem(b, 2), :, pl.ds(s * _CS, _CS)],
            out_hbm.at[b, :, pl.ds(s * _CS, _CS)],
            out_sems.at[jax.lax.rem(b, 2), s])

    @pl.when(t == 0)
    def _():
        for s in range(_S):
            in_copy(0, s).start()

    @pl.when(t < nb)
    def _():
        @pl.when(t + 1 < nb)
        def _():
            for s in range(_S):
                in_copy(t + 1, s).start()

        for s in range(_S):
            in_copy(t, s).wait()

        @pl.when(t >= 2)
        def _():
            for s in range(_S):
                out_copy(t - 2, s).wait()

        _compute(zbuf, obuf, jax.lax.rem(t, 2), cb_ref[...])
        for s in range(_S):
            out_copy(t, s).start()

    @pl.when(t == nb)
    def _():
        for s in range(_S):
            out_copy(nb - 2, s).wait()
            out_copy(nb - 1, s).wait()


def kernel(z, codebook):
    B, C, H, W = z.shape
    z3 = z.reshape(B, C, H * W)
    out = pl.pallas_call(
        _vq_body,
        grid=(B + 1,),
        in_specs=[
            pl.BlockSpec(memory_space=pl.ANY),
            pl.BlockSpec((_N_E, _E_DIM), lambda t: (0, 0)),
        ],
        out_specs=pl.BlockSpec(memory_space=pl.ANY),
        out_shape=jax.ShapeDtypeStruct((B, C, H * W), jnp.float32),
        scratch_shapes=[
            pltpu.VMEM((2, C, H * W), jnp.float32),
            pltpu.VMEM((2, C, H * W), jnp.float32),
            pltpu.SemaphoreType.DMA((2, _S)),
            pltpu.SemaphoreType.DMA((2, _S)),
        ],
        compiler_params=pltpu.CompilerParams(
            dimension_semantics=("arbitrary",),
        ),
    )(z3, codebook)
    return out.reshape(B, C, H, W)


# speedup vs baseline: 1.0354x; 1.0354x over previous
import jax
import jax.numpy as jnp
from jax.experimental import pallas as pl
from jax.experimental.pallas import tpu as pltpu

_N_E = 1024
_E_DIM = 256
_P = 1024


def _vq_body(z_ref, cb_ref, out_ref):
    z_b = z_ref[0]
    cb = cb_ref[...]
    znorm = jnp.sum(z_b * z_b, axis=0, keepdims=True)
    enorm = jnp.sum(cb * cb, axis=1, keepdims=True)
    mm2 = jax.lax.dot_general(
        cb * -2.0, z_b, (((1,), (0,)), ((), ())),
        preferred_element_type=jnp.float32)
    d = (znorm + enorm) + mm2
    iota_f = jax.lax.broadcasted_iota(jnp.int32, (_N_E, 1), 0).astype(jnp.float32)
    dmin = jnp.min(d, axis=0, keepdims=True)
    ikey = jnp.where(d == dmin, iota_f, float(_N_E))
    idx = jnp.min(ikey, axis=0, keepdims=True)
    onehot = (ikey == idx).astype(jnp.float32)
    out = jax.lax.dot_general(
        cb, onehot, (((0,), (0,)), ((), ())),
        preferred_element_type=jnp.float32)
    out_ref[0] = out


def kernel(z, codebook):
    B, C, H, W = z.shape
    z3 = z.reshape(B, C, H * W)
    out = pl.pallas_call(
        _vq_body,
        grid=(B,),
        in_specs=[
            pl.BlockSpec((1, C, H * W), lambda b: (b, 0, 0)),
            pl.BlockSpec((_N_E, _E_DIM), lambda b: (0, 0)),
        ],
        out_specs=pl.BlockSpec((1, C, H * W), lambda b: (b, 0, 0)),
        out_shape=jax.ShapeDtypeStruct((B, C, H * W), jnp.float32),
        compiler_params=pltpu.CompilerParams(
            dimension_semantics=("arbitrary",),
        ),
    )(z3, codebook)
    return out.reshape(B, C, H, W)


# speedup vs baseline: 1.0540x; 1.0180x over previous
import jax
import jax.numpy as jnp
from jax.experimental import pallas as pl
from jax.experimental.pallas import tpu as pltpu

_N_E = 1024
_E_DIM = 256
_P = 1024
_G = 4


def _vq_body(z_ref, cb_ref, out_ref):
    cb = cb_ref[...]
    enorm = jnp.sum(cb * cb, axis=1, keepdims=True)
    cb2 = cb * -2.0
    iota_f = jax.lax.broadcasted_iota(jnp.int32, (_N_E, 1), 0).astype(jnp.float32)
    for i in range(_G):
        z_b = z_ref[i]
        znorm = jnp.sum(z_b * z_b, axis=0, keepdims=True)
        mm2 = jax.lax.dot_general(
            cb2, z_b, (((1,), (0,)), ((), ())),
            preferred_element_type=jnp.float32)
        d = (znorm + enorm) + mm2
        dmin = jnp.min(d, axis=0, keepdims=True)
        ikey = jnp.where(d == dmin, iota_f, float(_N_E))
        idx = jnp.min(ikey, axis=0, keepdims=True)
        onehot = (ikey == idx).astype(jnp.float32)
        out_ref[i] = jax.lax.dot_general(
            cb, onehot, (((0,), (0,)), ((), ())),
            preferred_element_type=jnp.float32)


def kernel(z, codebook):
    B, C, H, W = z.shape
    z3 = z.reshape(B, C, H * W)
    out = pl.pallas_call(
        _vq_body,
        grid=(B // _G,),
        in_specs=[
            pl.BlockSpec((_G, C, H * W), lambda b: (b, 0, 0)),
            pl.BlockSpec((_N_E, _E_DIM), lambda b: (0, 0)),
        ],
        out_specs=pl.BlockSpec((_G, C, H * W), lambda b: (b, 0, 0)),
        out_shape=jax.ShapeDtypeStruct((B, C, H * W), jnp.float32),
        compiler_params=pltpu.CompilerParams(
            dimension_semantics=("parallel",),
        ),
    )(z3, codebook)
    return out.reshape(B, C, H, W)
